# Initial kernel scaffold; baseline (speedup 1.0000x reference)
import jax
import jax.numpy as jnp
from jax.experimental import pallas as pl


def kernel(input_tensor, edge_index1, weights1, bias1, edge_index2, weights2, bias2):
    raise NotImplementedError("write your pallas kernel here")


# speedup vs baseline: 22.5915x; 22.5915x over previous
import functools

import jax
import jax.numpy as jnp
from jax import lax
from jax.experimental import pallas as pl
from jax.experimental.pallas import tpu as pltpu
from jax.experimental.pallas import tpu_sc as plsc

L = 1024
E = 65536
NC, NS = 2, 16
NW = NC * NS
EPT = E // NW
CHUNK = 128
NCHUNK = EPT // CHUNK
SLICE = (L * L) // NS
ZB = 8192


def _build_w_body(dst_hbm, src_hbm, w_hbm, out_hbm,
                  dst_v, src_v, val_v, idx_v, zbuf_v, w_sh):
    c = lax.axis_index("c")
    s = lax.axis_index("s")
    wid = c * NS + s
    base = wid * EPT

    pltpu.sync_copy(dst_hbm.at[pl.ds(base, EPT)], dst_v)
    pltpu.sync_copy(src_hbm.at[pl.ds(base, EPT)], src_v)
    pltpu.sync_copy(w_hbm.at[pl.ds(base, EPT)], val_v)

    for k in range(ZB // 16):
        zbuf_v[pl.ds(k * 16, 16)] = jnp.zeros((16,), jnp.float32)
    for k in range(SLICE // ZB):
        pltpu.sync_copy(zbuf_v, w_sh.at[pl.ds(s * SLICE + k * ZB, ZB)])

    for j in range(NCHUNK):
        for i in range(CHUNK // 16):
            t = (j * CHUNK + i * 16)
            flat = src_v[pl.ds(t, 16)] * L + dst_v[pl.ds(t, 16)]
            idx_v[j, pl.ds(i * 16, 16)] = flat

    plsc.subcore_barrier()

    for j in range(NCHUNK):
        pltpu.sync_copy(val_v.at[pl.ds(j * CHUNK, CHUNK)],
                        w_sh.at[idx_v.at[j]], add=True)

    plsc.subcore_barrier()

    pltpu.sync_copy(w_sh.at[pl.ds(s * SLICE, SLICE)],
                    out_hbm.at[c, pl.ds(s * SLICE, SLICE)])


@functools.partial(
    pl.kernel,
    out_type=jax.ShapeDtypeStruct((NC, L * L), jnp.float32),
    mesh=plsc.VectorSubcoreMesh(core_axis_name="c", subcore_axis_name="s"),
    scratch_types=[
        pltpu.VMEM((EPT,), jnp.int32),
        pltpu.VMEM((EPT,), jnp.int32),
        pltpu.VMEM((EPT,), jnp.float32),
        pltpu.VMEM((NCHUNK, CHUNK), jnp.int32),
        pltpu.VMEM((ZB,), jnp.float32),
        pltpu.VMEM_SHARED((L * L,), jnp.float32),
    ],
)
def _build_w(dst_hbm, src_hbm, w_hbm, out_hbm,
             dst_v, src_v, val_v, idx_v, zbuf_v, w_sh):
    _build_w_body(dst_hbm, src_hbm, w_hbm, out_hbm,
                  dst_v, src_v, val_v, idx_v, zbuf_v, w_sh)


def _mlp_body(x_ref, w1_ref, b1_ref, w2_ref, b2_ref, o_ref):
    w1 = w1_ref[0] + w1_ref[1]
    h = jnp.dot(x_ref[...], w1, preferred_element_type=jnp.float32)
    h = jnp.maximum(h + b1_ref[...], 0.0)
    w2 = w2_ref[0] + w2_ref[1]
    o_ref[...] = jnp.dot(h, w2, preferred_element_type=jnp.float32) + b2_ref[...]


def kernel(input_tensor, edge_index1, weights1, bias1,
           edge_index2, weights2, bias2):
    d1 = edge_index1[0].astype(jnp.int32)
    s1 = edge_index1[1].astype(jnp.int32)
    d2 = edge_index2[0].astype(jnp.int32)
    s2 = edge_index2[1].astype(jnp.int32)

    w1p = _build_w(d1, s1, weights1).reshape(NC, L, L)
    w2p = _build_w(d2, s2, weights2).reshape(NC, L, L)

    out = pl.pallas_call(
        _mlp_body,
        out_shape=jax.ShapeDtypeStruct((input_tensor.shape[0], L), jnp.float32),
    )(input_tensor, w1p, bias1.reshape(1, L), w2p, bias2.reshape(1, L))
    return out


# speedup vs baseline: 29.7839x; 1.3184x over previous
import functools

import jax
import jax.numpy as jnp
from jax import lax
from jax.experimental import pallas as pl
from jax.experimental.pallas import tpu as pltpu
from jax.experimental.pallas import tpu_sc as plsc

L = 1024
E = 65536
NC, NS = 2, 16
EPC = E // NS
CHUNK = 128
NCHUNK = EPC // CHUNK
SLICE = (L * L) // NS
ZB = 8192


def _build_w_body(dst_hbm, src_hbm, w_hbm, out_hbm,
                  dst_v, src_v, val_v, idx_v, zbuf_v, w_sh):
    c = lax.axis_index("c")
    s = lax.axis_index("s")
    base = c * E + s * EPC

    pltpu.sync_copy(dst_hbm.at[pl.ds(base, EPC)], dst_v)
    pltpu.sync_copy(src_hbm.at[pl.ds(base, EPC)], src_v)
    pltpu.sync_copy(w_hbm.at[pl.ds(base, EPC)], val_v)

    for k in range(ZB // 16):
        zbuf_v[pl.ds(k * 16, 16)] = jnp.zeros((16,), jnp.float32)
    for k in range(SLICE // ZB):
        pltpu.sync_copy(zbuf_v, w_sh.at[pl.ds(s * SLICE + k * ZB, ZB)])

    for j in range(NCHUNK):
        for i in range(CHUNK // 16):
            t = j * CHUNK + i * 16
            idx_v[j, pl.ds(i * 16, 16)] = (src_v[pl.ds(t, 16)] * L
                                           + dst_v[pl.ds(t, 16)])
    plsc.subcore_barrier()

    for j in range(NCHUNK):
        pltpu.sync_copy(val_v.at[pl.ds(j * CHUNK, CHUNK)],
                        w_sh.at[idx_v.at[j]], add=True)
    plsc.subcore_barrier()

    pltpu.sync_copy(w_sh.at[pl.ds(s * SLICE, SLICE)],
                    out_hbm.at[c, pl.ds(s * SLICE, SLICE)])


@functools.partial(
    pl.kernel,
    out_type=jax.ShapeDtypeStruct((NC, L * L), jnp.float32),
    mesh=plsc.VectorSubcoreMesh(core_axis_name="c", subcore_axis_name="s"),
    scratch_types=[
        pltpu.VMEM((EPC,), jnp.int32),
        pltpu.VMEM((EPC,), jnp.int32),
        pltpu.VMEM((EPC,), jnp.float32),
        pltpu.VMEM((NCHUNK, CHUNK), jnp.int32),
        pltpu.VMEM((ZB,), jnp.float32),
        pltpu.VMEM_SHARED((L * L,), jnp.float32),
    ],
)
def _build_w(dst_hbm, src_hbm, w_hbm, out_hbm,
             dst_v, src_v, val_v, idx_v, zbuf_v, w_sh):
    _build_w_body(dst_hbm, src_hbm, w_hbm, out_hbm,
                  dst_v, src_v, val_v, idx_v, zbuf_v, w_sh)


def _mlp_body(x_ref, w_ref, b1_ref, b2_ref, o_ref):
    h = jnp.dot(x_ref[...], w_ref[0], preferred_element_type=jnp.float32)
    h = jnp.maximum(h + b1_ref[...], 0.0)
    o_ref[...] = jnp.dot(h, w_ref[1], preferred_element_type=jnp.float32) + b2_ref[...]


def kernel(input_tensor, edge_index1, weights1, bias1,
           edge_index2, weights2, bias2):
    dst_all = jnp.concatenate([edge_index1[0], edge_index2[0]]).astype(jnp.int32)
    src_all = jnp.concatenate([edge_index1[1], edge_index2[1]]).astype(jnp.int32)
    w_all = jnp.concatenate([weights1, weights2])

    w12 = _build_w(dst_all, src_all, w_all).reshape(NC, L, L)

    out = pl.pallas_call(
        _mlp_body,
        out_shape=jax.ShapeDtypeStruct((input_tensor.shape[0], L), jnp.float32),
    )(input_tensor, w12, bias1.reshape(1, L), bias2.reshape(1, L))
    return out
